# baseline (speedup 1.0000x reference)
import functools

import jax
import jax.numpy as jnp
from jax import lax
from jax.experimental import pallas as pl
from jax.experimental.pallas import tpu as pltpu
from jax.experimental.pallas import tpu_sc as plsc

N_ROWS = 100000
D = 128
NUM_GRAPHS = 64
REUSE_WEIGHT = 0.01

TCB = 1024
NTCB = 65
TC_ROWS = NTCB * TCB
SC_ROWS = N_ROWS - TC_ROWS

NW = 32
CHUNK = 128
CPW = (SC_ROWS + NW * CHUNK - 1) // (NW * CHUNK)
N_CHUNKS = NW * CPW
ACC_ROWS = 72
CW = 16
SC_LAST = SC_ROWS - CHUNK


def _sc_partial_sums(codes, idx3d, zeros_s, zeros_c, ones_c):
    mesh = plsc.VectorSubcoreMesh(core_axis_name="c", subcore_axis_name="s")

    @functools.partial(
        pl.kernel,
        mesh=mesh,
        out_type=[
            jax.ShapeDtypeStruct((2, ACC_ROWS, D), jnp.float32),
            jax.ShapeDtypeStruct((2, ACC_ROWS, CW), jnp.float32),
        ],
        scratch_types=[
            pltpu.VMEM((1, CPW, CHUNK), jnp.int32),
            pltpu.VMEM((2, CHUNK, D), jnp.float32),
            pltpu.VMEM((CHUNK, CW), jnp.float32),
            pltpu.VMEM_SHARED((ACC_ROWS, D), jnp.float32),
            pltpu.VMEM_SHARED((ACC_ROWS, CW), jnp.float32),
            pltpu.SemaphoreType.DMA,
            pltpu.SemaphoreType.DMA,
        ],
    )
    def sc_kernel(codes_hbm, idx_hbm, zs_hbm, zc_hbm, ones_hbm,
                  out_s_hbm, out_c_hbm,
                  idx_v, stage_v, ones_v, acc_sh, cnt_sh, sem0, sem1):
        cid = lax.axis_index("c")
        sid = lax.axis_index("s")
        wid = cid * 16 + sid
        sems = (sem0, sem1)

        @pl.when(sid == 0)
        def _():
            pltpu.sync_copy(zs_hbm, acc_sh)
            pltpu.sync_copy(zc_hbm, cnt_sh)

        pltpu.sync_copy(ones_hbm, ones_v)
        pltpu.sync_copy(idx_hbm.at[pl.ds(wid, 1)], idx_v)
        plsc.subcore_barrier()

        def start_in(c, b):
            src = TC_ROWS + jnp.minimum((wid * CPW + c) * CHUNK, SC_LAST)
            pltpu.async_copy(codes_hbm.at[pl.ds(src, CHUNK)], stage_v.at[b],
                             sems[b])

        def wait_in(b):
            pltpu.make_async_copy(codes_hbm.at[pl.ds(0, CHUNK)],
                                  stage_v.at[b], sems[b]).wait()

        def scatter(c, b):
            pltpu.sync_copy(stage_v.at[b], acc_sh.at[idx_v.at[0, c]],
                            add=True)
            pltpu.sync_copy(ones_v, cnt_sh.at[idx_v.at[0, c]], add=True)

        start_in(0, 0)

        def body(i, carry):
            base = i * 2
            wait_in(0)
            start_in(base + 1, 1)
            scatter(base, 0)
            wait_in(1)
            start_in(base + 2, 0)
            scatter(base + 1, 1)
            return carry

        lax.fori_loop(0, (CPW - 1) // 2, body, 0)
        wait_in(0)
        scatter(CPW - 1, 0)
        plsc.subcore_barrier()

        @pl.when(sid == 0)
        def _():
            pltpu.sync_copy(acc_sh, out_s_hbm.at[cid])
            pltpu.sync_copy(cnt_sh, out_c_hbm.at[cid])

    return sc_kernel(codes, idx3d, zeros_s, zeros_c, ones_c)


def _tc_partial_sums(codes, batch3d):
    def body(b_ref, x_ref, s_ref, c_ref):
        g = pl.program_id(0)
        b = b_ref[0, 0:1, :]
        iot = lax.broadcasted_iota(jnp.int32, (NUM_GRAPHS, TCB), 0)
        oh = (jnp.broadcast_to(b, (NUM_GRAPHS, TCB)) == iot)
        ohf = oh.astype(jnp.float32)
        ps = jax.lax.dot_general(ohf, x_ref[...], (((1,), (0,)), ((), ())),
                                 preferred_element_type=jnp.float32)
        pc = jnp.sum(ohf, axis=1, keepdims=True)

        @pl.when(g == 0)
        def _():
            s_ref[...] = jnp.zeros_like(s_ref)
            c_ref[...] = jnp.zeros_like(c_ref)

        s_ref[...] += ps
        c_ref[...] += jnp.broadcast_to(pc, (NUM_GRAPHS, D))

    return pl.pallas_call(
        body,
        grid=(NTCB,),
        in_specs=[
            pl.BlockSpec((1, 1, TCB), lambda g: (g, 0, 0)),
            pl.BlockSpec((TCB, D), lambda g: (g, 0)),
        ],
        out_specs=[
            pl.BlockSpec((NUM_GRAPHS, D), lambda g: (0, 0)),
            pl.BlockSpec((NUM_GRAPHS, D), lambda g: (0, 0)),
        ],
        out_shape=[
            jax.ShapeDtypeStruct((NUM_GRAPHS, D), jnp.float32),
            jax.ShapeDtypeStruct((NUM_GRAPHS, D), jnp.float32),
        ],
    )(batch3d, codes)


def _tc_finalize(sc_s, sc_c, tc_s, tc_c):
    def body(ss_ref, sc_ref, ts_ref, tcn_ref, o_ref):
        s = ss_ref[0, :NUM_GRAPHS, :] + ss_ref[1, :NUM_GRAPHS, :] + ts_ref[...]
        cnt = (sc_ref[0, :NUM_GRAPHS, 0:1] + sc_ref[1, :NUM_GRAPHS, 0:1]
               + tcn_ref[:, 0:1])
        m = s / cnt
        mu = jnp.mean(m, axis=0, keepdims=True)
        var = jnp.sum((m - mu) ** 2, axis=0) / (NUM_GRAPHS - 1)
        o_ref[0, 0] = -REUSE_WEIGHT * jnp.mean(var)

    return pl.pallas_call(
        body,
        out_shape=jax.ShapeDtypeStruct((1, 1), jnp.float32),
        out_specs=pl.BlockSpec(memory_space=pltpu.SMEM),
    )(sc_s, sc_c, tc_s, tc_c)


def kernel(sparse_codes, batch):
    b32 = batch.astype(jnp.int32)

    b_sfx = b32[TC_ROWS:]
    q = jnp.arange(N_CHUNKS, dtype=jnp.int32)
    src = jnp.minimum(q * CHUNK, SC_LAST)
    grow = src[:, None] + jnp.arange(CHUNK, dtype=jnp.int32)[None, :]
    seg = jnp.where(grow >= q[:, None] * CHUNK, b_sfx[grow],
                    jnp.int32(NUM_GRAPHS))
    idx3d = seg.reshape(NW, CPW, CHUNK)

    zeros_s = jnp.zeros((ACC_ROWS, D), jnp.float32)
    zeros_c = jnp.zeros((ACC_ROWS, CW), jnp.float32)
    ones_c = jnp.ones((CHUNK, CW), jnp.float32)

    batch3d = b32[:TC_ROWS].reshape(NTCB, 1, TCB)

    sc_s, sc_c = _sc_partial_sums(sparse_codes, idx3d, zeros_s, zeros_c,
                                  ones_c)
    tc_s, tc_c = _tc_partial_sums(sparse_codes, batch3d)
    return _tc_finalize(sc_s, sc_c, tc_s, tc_c)[0, 0]


# speedup vs baseline: 1.8773x; 1.8773x over previous
import functools

import jax
import jax.numpy as jnp
from jax import lax
from jax.experimental import pallas as pl
from jax.experimental.pallas import tpu as pltpu
from jax.experimental.pallas import tpu_sc as plsc

N_ROWS = 100000
D = 128
NUM_GRAPHS = 64
REUSE_WEIGHT = 0.01

TCB = 4096
NTCB = 16
TC_ROWS = NTCB * TCB
SC_ROWS = N_ROWS - TC_ROWS

NW = 32
CHUNK = 128
CPW = (SC_ROWS + NW * CHUNK - 1) // (NW * CHUNK)
N_CHUNKS = NW * CPW
N_FULL = SC_ROWS // CHUNK
ACC_ROWS = 72
CW = 16
SC_LAST = SC_ROWS - CHUNK


def _sc_partial_sums(codes, idx3d, zeros_s, zeros_c, ones_c):
    mesh = plsc.VectorSubcoreMesh(core_axis_name="c", subcore_axis_name="s")

    @functools.partial(
        pl.kernel,
        mesh=mesh,
        out_type=[
            jax.ShapeDtypeStruct((2, ACC_ROWS, D), jnp.float32),
            jax.ShapeDtypeStruct((2, ACC_ROWS, CW), jnp.float32),
        ],
        scratch_types=[
            pltpu.VMEM((1, CPW, CHUNK), jnp.int32),
            pltpu.VMEM((2, CHUNK, D), jnp.float32),
            pltpu.VMEM((CHUNK, CW), jnp.float32),
            pltpu.VMEM_SHARED((ACC_ROWS, D), jnp.float32),
            pltpu.VMEM_SHARED((ACC_ROWS, CW), jnp.float32),
            pltpu.SemaphoreType.DMA,
            pltpu.SemaphoreType.DMA,
        ],
    )
    def sc_kernel(codes_hbm, idx_hbm, zs_hbm, zc_hbm, ones_hbm,
                  out_s_hbm, out_c_hbm,
                  idx_v, stage_v, ones_v, acc_sh, cnt_sh, sem0, sem1):
        cid = lax.axis_index("c")
        sid = lax.axis_index("s")
        wid = cid * 16 + sid
        sems = (sem0, sem1)

        @pl.when(sid == 0)
        def _():
            pltpu.sync_copy(zs_hbm, acc_sh)
            pltpu.sync_copy(zc_hbm, cnt_sh)

        pltpu.sync_copy(ones_hbm, ones_v)
        pltpu.sync_copy(idx_hbm.at[pl.ds(wid, 1)], idx_v)
        plsc.subcore_barrier()

        def start_in(c, b):
            src = TC_ROWS + jnp.minimum((wid * CPW + c) * CHUNK, SC_LAST)
            pltpu.async_copy(codes_hbm.at[pl.ds(src, CHUNK)], stage_v.at[b],
                             sems[b])

        def wait_in(b):
            pltpu.make_async_copy(codes_hbm.at[pl.ds(0, CHUNK)],
                                  stage_v.at[b], sems[b]).wait()

        def scatter(c, b):
            pltpu.sync_copy(stage_v.at[b], acc_sh.at[idx_v.at[0, c]],
                            add=True)
            pltpu.sync_copy(ones_v, cnt_sh.at[idx_v.at[0, c]], add=True)

        start_in(0, 0)

        def body(i, carry):
            base = i * 2
            wait_in(0)
            start_in(base + 1, 1)
            scatter(base, 0)
            wait_in(1)
            start_in(base + 2, 0)
            scatter(base + 1, 1)
            return carry

        lax.fori_loop(0, (CPW - 1) // 2, body, 0)
        wait_in(0)
        scatter(CPW - 1, 0)
        plsc.subcore_barrier()

        @pl.when(sid == 0)
        def _():
            pltpu.sync_copy(acc_sh, out_s_hbm.at[cid])
            pltpu.sync_copy(cnt_sh, out_c_hbm.at[cid])

    return sc_kernel(codes, idx3d, zeros_s, zeros_c, ones_c)


def _tc_partial_sums(codes, batch3d):
    def body(b_ref, x_ref, s_ref, c_ref):
        g = pl.program_id(0)
        b = b_ref[0, 0:1, :]
        iot = lax.broadcasted_iota(jnp.int32, (NUM_GRAPHS, TCB), 0)
        oh = (jnp.broadcast_to(b, (NUM_GRAPHS, TCB)) == iot)
        ohf = oh.astype(jnp.float32)
        ps = jax.lax.dot_general(ohf, x_ref[...], (((1,), (0,)), ((), ())),
                                 preferred_element_type=jnp.float32)
        pc = jnp.sum(ohf, axis=1, keepdims=True)

        @pl.when(g == 0)
        def _():
            s_ref[...] = jnp.zeros_like(s_ref)
            c_ref[...] = jnp.zeros_like(c_ref)

        s_ref[...] += ps
        c_ref[...] += jnp.broadcast_to(pc, (NUM_GRAPHS, D))

    return pl.pallas_call(
        body,
        grid=(NTCB,),
        in_specs=[
            pl.BlockSpec((1, 1, TCB), lambda g: (g, 0, 0)),
            pl.BlockSpec((TCB, D), lambda g: (g, 0)),
        ],
        out_specs=[
            pl.BlockSpec((NUM_GRAPHS, D), lambda g: (0, 0)),
            pl.BlockSpec((NUM_GRAPHS, D), lambda g: (0, 0)),
        ],
        out_shape=[
            jax.ShapeDtypeStruct((NUM_GRAPHS, D), jnp.float32),
            jax.ShapeDtypeStruct((NUM_GRAPHS, D), jnp.float32),
        ],
    )(batch3d, codes)


def _tc_finalize(sc_s, sc_c, tc_s, tc_c):
    def body(ss_ref, sc_ref, ts_ref, tcn_ref, o_ref):
        s = ss_ref[0, :NUM_GRAPHS, :] + ss_ref[1, :NUM_GRAPHS, :] + ts_ref[...]
        cnt = (sc_ref[0, :NUM_GRAPHS, 0:1] + sc_ref[1, :NUM_GRAPHS, 0:1]
               + tcn_ref[:, 0:1])
        m = s / cnt
        mu = jnp.mean(m, axis=0, keepdims=True)
        var = jnp.sum((m - mu) ** 2, axis=0) / (NUM_GRAPHS - 1)
        o_ref[0, 0] = -REUSE_WEIGHT * jnp.mean(var)

    return pl.pallas_call(
        body,
        out_shape=jax.ShapeDtypeStruct((1, 1), jnp.float32),
        out_specs=pl.BlockSpec(memory_space=pltpu.SMEM),
    )(sc_s, sc_c, tc_s, tc_c)


def kernel(sparse_codes, batch):
    b32 = batch.astype(jnp.int32)

    b_sfx = b32[TC_ROWS:]
    idx_main = b_sfx[: N_FULL * CHUNK].reshape(N_FULL, CHUNK)
    tail_valid = SC_ROWS - N_FULL * CHUNK
    idx_tail = jnp.concatenate(
        [jnp.full((CHUNK - tail_valid,), NUM_GRAPHS, jnp.int32),
         b_sfx[N_FULL * CHUNK:]]).reshape(1, CHUNK)
    idx_pad = jnp.full((N_CHUNKS - N_FULL - 1, CHUNK), NUM_GRAPHS, jnp.int32)
    idx3d = jnp.concatenate([idx_main, idx_tail, idx_pad],
                            axis=0).reshape(NW, CPW, CHUNK)

    zeros_s = jnp.zeros((ACC_ROWS, D), jnp.float32)
    zeros_c = jnp.zeros((ACC_ROWS, CW), jnp.float32)
    ones_c = jnp.ones((CHUNK, CW), jnp.float32)

    batch3d = b32[:TC_ROWS].reshape(NTCB, 1, TCB)

    sc_s, sc_c = _sc_partial_sums(sparse_codes, idx3d, zeros_s, zeros_c,
                                  ones_c)
    tc_s, tc_c = _tc_partial_sums(sparse_codes, batch3d)
    return _tc_finalize(sc_s, sc_c, tc_s, tc_c)[0, 0]


# speedup vs baseline: 1.9509x; 1.0392x over previous
import functools

import numpy as np
import jax
import jax.numpy as jnp
from jax import lax
from jax.experimental import pallas as pl
from jax.experimental.pallas import tpu as pltpu
from jax.experimental.pallas import tpu_sc as plsc

N_ROWS = 100000
D = 128
NUM_GRAPHS = 64
REUSE_WEIGHT = 0.01

NW = 32
CHUNK = 128
CPW = 8
SC_ROWS = NW * CPW * CHUNK
TC_ROWS = N_ROWS - SC_ROWS
NTCB = 11
TCB = TC_ROWS // NTCB
ACC_ROWS = 72
CW = 16


def _sc_partial_sums(codes, idx3d, zeros_s, zeros_c, ones_c):
    mesh = plsc.VectorSubcoreMesh(core_axis_name="c", subcore_axis_name="s")

    @functools.partial(
        pl.kernel,
        mesh=mesh,
        out_type=[
            jax.ShapeDtypeStruct((2, ACC_ROWS, D), jnp.float32),
            jax.ShapeDtypeStruct((2, ACC_ROWS, CW), jnp.float32),
        ],
        scratch_types=[
            pltpu.VMEM((1, CPW, CHUNK), jnp.int32),
            pltpu.VMEM((2, CHUNK, D), jnp.float32),
            pltpu.VMEM((CHUNK, CW), jnp.float32),
            pltpu.VMEM_SHARED((ACC_ROWS, D), jnp.float32),
            pltpu.VMEM_SHARED((ACC_ROWS, CW), jnp.float32),
            pltpu.SemaphoreType.DMA,
            pltpu.SemaphoreType.DMA,
        ],
    )
    def sc_kernel(codes_hbm, idx_hbm, zs_hbm, zc_hbm, ones_hbm,
                  out_s_hbm, out_c_hbm,
                  idx_v, stage_v, ones_v, acc_sh, cnt_sh, sem0, sem1):
        cid = lax.axis_index("c")
        sid = lax.axis_index("s")
        wid = cid * 16 + sid
        sems = (sem0, sem1)

        @pl.when(sid == 0)
        def _():
            pltpu.sync_copy(zs_hbm, acc_sh)
            pltpu.sync_copy(zc_hbm, cnt_sh)

        pltpu.sync_copy(ones_hbm, ones_v)
        pltpu.sync_copy(idx_hbm.at[pl.ds(wid, 1)], idx_v)
        plsc.subcore_barrier()

        def start_in(c, b):
            src = TC_ROWS + (wid * CPW + c) * CHUNK
            pltpu.async_copy(codes_hbm.at[pl.ds(src, CHUNK)], stage_v.at[b],
                             sems[b])

        def wait_in(b):
            pltpu.make_async_copy(codes_hbm.at[pl.ds(0, CHUNK)],
                                  stage_v.at[b], sems[b]).wait()

        def scatter(c, b):
            pltpu.sync_copy(stage_v.at[b], acc_sh.at[idx_v.at[0, c]],
                            add=True)
            pltpu.sync_copy(ones_v, cnt_sh.at[idx_v.at[0, c]], add=True)

        start_in(0, 0)

        def body(i, carry):
            base = i * 2
            wait_in(0)
            start_in(base + 1, 1)
            scatter(base, 0)
            wait_in(1)
            start_in(base + 2, 0)
            scatter(base + 1, 1)
            return carry

        lax.fori_loop(0, (CPW - 2) // 2, body, 0)
        wait_in(0)
        start_in(CPW - 1, 1)
        scatter(CPW - 2, 0)
        wait_in(1)
        scatter(CPW - 1, 1)
        plsc.subcore_barrier()

        @pl.when(sid == 0)
        def _():
            pltpu.sync_copy(acc_sh, out_s_hbm.at[cid])
            pltpu.sync_copy(cnt_sh, out_c_hbm.at[cid])

    return sc_kernel(codes, idx3d, zeros_s, zeros_c, ones_c)


def _tc_partial_sums(codes, batch3d):
    def body(b_ref, x_ref, s_ref, c_ref):
        g = pl.program_id(0)
        b = b_ref[0, 0:1, :]
        iot = lax.broadcasted_iota(jnp.int32, (NUM_GRAPHS, TCB), 0)
        oh = (jnp.broadcast_to(b, (NUM_GRAPHS, TCB)) == iot)
        ohf = oh.astype(jnp.float32)
        ps = jax.lax.dot_general(ohf, x_ref[...], (((1,), (0,)), ((), ())),
                                 preferred_element_type=jnp.float32)
        pc = jnp.sum(ohf, axis=1, keepdims=True)

        @pl.when(g == 0)
        def _():
            s_ref[...] = jnp.zeros_like(s_ref)
            c_ref[...] = jnp.zeros_like(c_ref)

        s_ref[...] += ps
        c_ref[...] += jnp.broadcast_to(pc, (NUM_GRAPHS, D))

    return pl.pallas_call(
        body,
        grid=(NTCB,),
        in_specs=[
            pl.BlockSpec((1, 1, TCB), lambda g: (g, 0, 0)),
            pl.BlockSpec((TCB, D), lambda g: (g, 0)),
        ],
        out_specs=[
            pl.BlockSpec((NUM_GRAPHS, D), lambda g: (0, 0)),
            pl.BlockSpec((NUM_GRAPHS, D), lambda g: (0, 0)),
        ],
        out_shape=[
            jax.ShapeDtypeStruct((NUM_GRAPHS, D), jnp.float32),
            jax.ShapeDtypeStruct((NUM_GRAPHS, D), jnp.float32),
        ],
    )(batch3d, codes)


def _tc_finalize(sc_s, sc_c, tc_s, tc_c):
    def body(ss_ref, sc_ref, ts_ref, tcn_ref, o_ref):
        s = ss_ref[0, :NUM_GRAPHS, :] + ss_ref[1, :NUM_GRAPHS, :] + ts_ref[...]
        cnt = (sc_ref[0, :NUM_GRAPHS, 0:1] + sc_ref[1, :NUM_GRAPHS, 0:1]
               + tcn_ref[:, 0:1])
        m = s / cnt
        mu = jnp.mean(m, axis=0, keepdims=True)
        var = jnp.sum((m - mu) ** 2, axis=0) / (NUM_GRAPHS - 1)
        o_ref[0, 0] = -REUSE_WEIGHT * jnp.mean(var)

    return pl.pallas_call(
        body,
        out_shape=jax.ShapeDtypeStruct((1, 1), jnp.float32),
        out_specs=pl.BlockSpec(memory_space=pltpu.SMEM),
    )(sc_s, sc_c, tc_s, tc_c)


def kernel(sparse_codes, batch):
    b32 = batch.astype(jnp.int32)

    idx3d = b32[TC_ROWS:].reshape(NW, CPW, CHUNK)
    batch3d = b32[:TC_ROWS].reshape(NTCB, 1, TCB)

    zeros_s = np.zeros((ACC_ROWS, D), np.float32)
    zeros_c = np.zeros((ACC_ROWS, CW), np.float32)
    ones_c = np.ones((CHUNK, CW), np.float32)

    sc_s, sc_c = _sc_partial_sums(sparse_codes, idx3d, zeros_s, zeros_c,
                                  ones_c)
    tc_s, tc_c = _tc_partial_sums(sparse_codes, batch3d)
    return _tc_finalize(sc_s, sc_c, tc_s, tc_c)[0, 0]


# speedup vs baseline: 2.2591x; 1.1580x over previous
import functools

import jax
import jax.numpy as jnp
from jax import lax
from jax.experimental import pallas as pl
from jax.experimental.pallas import tpu as pltpu
from jax.experimental.pallas import tpu_sc as plsc

N_ROWS = 100000
D = 128
NUM_GRAPHS = 64
REUSE_WEIGHT = 0.01

NW = 32
CHUNK = 128
CPW = 7
SC_ROWS = NW * CPW * CHUNK
TC_ROWS = N_ROWS - SC_ROWS
NTCB = 4
TCB = TC_ROWS // NTCB
ACC_ROWS = 72
CW = 16


def _sc_partial_sums(codes, idx3d):
    mesh = plsc.VectorSubcoreMesh(core_axis_name="c", subcore_axis_name="s")

    @functools.partial(
        pl.kernel,
        mesh=mesh,
        out_type=[
            jax.ShapeDtypeStruct((2, ACC_ROWS, D), jnp.float32),
            jax.ShapeDtypeStruct((2, ACC_ROWS, CW), jnp.float32),
        ],
        scratch_types=[
            pltpu.VMEM((1, CPW, CHUNK), jnp.int32),
            pltpu.VMEM((2, CHUNK, D), jnp.float32),
            pltpu.VMEM((CHUNK, CW), jnp.float32),
            pltpu.VMEM_SHARED((ACC_ROWS, D), jnp.float32),
            pltpu.VMEM_SHARED((ACC_ROWS, CW), jnp.float32),
            pltpu.SemaphoreType.DMA,
            pltpu.SemaphoreType.DMA,
        ],
    )
    def sc_kernel(codes_hbm, idx_hbm, out_s_hbm, out_c_hbm,
                  idx_v, stage_v, ones_v, acc_sh, cnt_sh, sem0, sem1):
        cid = lax.axis_index("c")
        sid = lax.axis_index("s")
        wid = cid * 16 + sid
        sems = (sem0, sem1)

        pltpu.sync_copy(idx_hbm.at[pl.ds(wid, 1)], idx_v)

        zv = jnp.zeros((16,), jnp.float32)
        ov = jnp.ones((16,), jnp.float32)

        @pl.when(sid == 0)
        def _():
            def zrow(r, carry):
                for c8 in range(D // 16):
                    stage_v[0, r, c8 * 16:(c8 + 1) * 16] = zv
                ones_v[r % CHUNK, :] = zv
                return carry

            lax.fori_loop(0, ACC_ROWS, zrow, 0)
            pltpu.sync_copy(stage_v.at[0, pl.ds(0, ACC_ROWS)], acc_sh)
            pltpu.sync_copy(ones_v.at[pl.ds(0, ACC_ROWS)], cnt_sh)

        def orow(r, carry):
            ones_v[r, :] = ov
            return carry

        lax.fori_loop(0, CHUNK, orow, 0)
        plsc.subcore_barrier()

        def start_in(c, b):
            src = TC_ROWS + (wid * CPW + c) * CHUNK
            pltpu.async_copy(codes_hbm.at[pl.ds(src, CHUNK)], stage_v.at[b],
                             sems[b])

        def wait_in(b):
            pltpu.make_async_copy(codes_hbm.at[pl.ds(0, CHUNK)],
                                  stage_v.at[b], sems[b]).wait()

        def scatter(c, b):
            pltpu.sync_copy(stage_v.at[b], acc_sh.at[idx_v.at[0, c]],
                            add=True)
            pltpu.sync_copy(ones_v, cnt_sh.at[idx_v.at[0, c]], add=True)

        start_in(0, 0)

        def body(i, carry):
            base = i * 2
            wait_in(0)
            start_in(base + 1, 1)
            scatter(base, 0)
            wait_in(1)
            start_in(base + 2, 0)
            scatter(base + 1, 1)
            return carry

        if CPW % 2:
            lax.fori_loop(0, (CPW - 1) // 2, body, 0)
            wait_in(0)
            scatter(CPW - 1, 0)
        else:
            lax.fori_loop(0, (CPW - 2) // 2, body, 0)
            wait_in(0)
            start_in(CPW - 1, 1)
            scatter(CPW - 2, 0)
            wait_in(1)
            scatter(CPW - 1, 1)
        plsc.subcore_barrier()

        @pl.when(sid == 0)
        def _():
            pltpu.sync_copy(acc_sh, out_s_hbm.at[cid])
            pltpu.sync_copy(cnt_sh, out_c_hbm.at[cid])

    return sc_kernel(codes, idx3d)


def _tc_partial_sums(codes, batch3d):
    def body(b_ref, x_ref, s_ref, c_ref):
        g = pl.program_id(0)
        b = b_ref[0, 0:1, :]
        iot = lax.broadcasted_iota(jnp.int32, (NUM_GRAPHS, TCB), 0)
        oh = (jnp.broadcast_to(b, (NUM_GRAPHS, TCB)) == iot)
        ohf = oh.astype(jnp.float32)
        ps = jax.lax.dot_general(ohf, x_ref[...], (((1,), (0,)), ((), ())),
                                 preferred_element_type=jnp.float32)
        pc = jnp.sum(ohf, axis=1, keepdims=True)

        @pl.when(g == 0)
        def _():
            s_ref[...] = jnp.zeros_like(s_ref)
            c_ref[...] = jnp.zeros_like(c_ref)

        s_ref[...] += ps
        c_ref[...] += jnp.broadcast_to(pc, (NUM_GRAPHS, D))

    return pl.pallas_call(
        body,
        grid=(NTCB,),
        in_specs=[
            pl.BlockSpec((1, 1, TCB), lambda g: (g, 0, 0)),
            pl.BlockSpec((TCB, D), lambda g: (g, 0)),
        ],
        out_specs=[
            pl.BlockSpec((NUM_GRAPHS, D), lambda g: (0, 0)),
            pl.BlockSpec((NUM_GRAPHS, D), lambda g: (0, 0)),
        ],
        out_shape=[
            jax.ShapeDtypeStruct((NUM_GRAPHS, D), jnp.float32),
            jax.ShapeDtypeStruct((NUM_GRAPHS, D), jnp.float32),
        ],
    )(batch3d, codes)


def _tc_finalize(sc_s, sc_c, tc_s, tc_c):
    def body(ss_ref, sc_ref, ts_ref, tcn_ref, o_ref):
        s = ss_ref[0, :NUM_GRAPHS, :] + ss_ref[1, :NUM_GRAPHS, :] + ts_ref[...]
        cnt = (sc_ref[0, :NUM_GRAPHS, 0:1] + sc_ref[1, :NUM_GRAPHS, 0:1]
               + tcn_ref[:, 0:1])
        m = s / cnt
        mu = jnp.mean(m, axis=0, keepdims=True)
        var = jnp.sum((m - mu) ** 2, axis=0) / (NUM_GRAPHS - 1)
        o_ref[0, 0] = -REUSE_WEIGHT * jnp.mean(var)

    return pl.pallas_call(
        body,
        out_shape=jax.ShapeDtypeStruct((1, 1), jnp.float32),
        out_specs=pl.BlockSpec(memory_space=pltpu.SMEM),
    )(sc_s, sc_c, tc_s, tc_c)


def kernel(sparse_codes, batch):
    b32 = batch.astype(jnp.int32)

    idx3d = b32[TC_ROWS:].reshape(NW, CPW, CHUNK)
    batch3d = b32[:TC_ROWS].reshape(NTCB, 1, TCB)

    sc_s, sc_c = _sc_partial_sums(sparse_codes, idx3d)
    tc_s, tc_c = _tc_partial_sums(sparse_codes, batch3d)
    return _tc_finalize(sc_s, sc_c, tc_s, tc_c)[0, 0]
